# baseline (speedup 1.0000x reference)
import functools

import jax
import jax.numpy as jnp
from jax import lax
from jax.experimental import pallas as pl
from jax.experimental.pallas import tpu as pltpu
from jax.experimental.pallas import tpu_sc as plsc

EMB = 64
VOCAB_ROWS = 129
CHUNK = 400
SUBS = ((0, 128), (128, 128), (256, 128), (384, 16))
PAIR_SPAN = 800
BI = 8


@functools.partial(jax.jit, static_argnames=("total",))
def _sc_embedding_gather(pair_table, pidx, total):
    info = plsc.get_sparse_core_info()
    num_workers = info.num_cores * info.num_subcores
    pairs_total = total // 2
    per_worker = pairs_total // num_workers
    n_chunks = per_worker // CHUNK
    half_t = n_chunks // 2
    mesh = plsc.VectorSubcoreMesh(core_axis_name="c", subcore_axis_name="s")

    @functools.partial(
        pl.kernel,
        mesh=mesh,
        out_type=jax.ShapeDtypeStruct((pairs_total, 2 * EMB), jnp.float32),
        scratch_types=[
            pltpu.VMEM((CHUNK,), jnp.int32),
            pltpu.VMEM((CHUNK,), jnp.int32),
            pltpu.VMEM((CHUNK, 2 * EMB), jnp.float32),
            pltpu.VMEM((CHUNK, 2 * EMB), jnp.float32),
            pltpu.SemaphoreType.DMA,
            pltpu.SemaphoreType.DMA,
            pltpu.SemaphoreType.DMA,
            pltpu.SemaphoreType.DMA,
        ],
    )
    def k(pt_hbm, pidx_hbm, out_hbm, pidx0, pidx1, rows0, rows1,
          gsem0, gsem1, osem0, osem1):
        wid = lax.axis_index("s") * info.num_cores + lax.axis_index("c")
        base = wid * per_worker

        def fire_gather(pidx_v, rows_v, gsem):
            for off, sz in SUBS:
                pltpu.async_copy(
                    pt_hbm.at[pidx_v.at[pl.ds(off, sz)]],
                    rows_v.at[pl.ds(off, sz)],
                    gsem,
                )

        def drain_gather(rows_v, gsem):
            pltpu.make_async_copy(out_hbm.at[pl.ds(0, CHUNK)], rows_v, gsem).wait()

        def drain_out(rows_v, osem):
            pltpu.make_async_copy(rows_v, out_hbm.at[pl.ds(0, CHUNK)], osem).wait()

        def load_idx(g, pidx_v):
            pltpu.sync_copy(pidx_hbm.at[pl.ds(base + g * CHUNK, CHUNK)], pidx_v)

        def fire_out(g, rows_v, osem):
            pltpu.async_copy(rows_v, out_hbm.at[pl.ds(base + g * CHUNK, CHUNK)], osem)

        load_idx(0, pidx0)
        fire_gather(pidx0, rows0, gsem0)

        def body(t, carry):
            g = 2 * t

            @pl.when(t > 0)
            def _():
                drain_out(rows1, osem1)

            load_idx(g + 1, pidx1)
            fire_gather(pidx1, rows1, gsem1)

            drain_gather(rows0, gsem0)
            fire_out(g, rows0, osem0)

            @pl.when(t < half_t - 1)
            def _():
                drain_out(rows0, osem0)
                load_idx(g + 2, pidx0)
                fire_gather(pidx0, rows0, gsem0)

            drain_gather(rows1, gsem1)
            fire_out(g + 1, rows1, osem1)
            return carry

        lax.fori_loop(0, half_t, body, 0)
        drain_out(rows0, osem0)
        drain_out(rows1, osem1)

    return k(pair_table, pidx)


def _tc_repack_block(in_ref, out_ref):
    x = in_ref[...]
    y = jnp.concatenate([x[:, :EMB], x[:, EMB:]], axis=0)
    out_ref[...] = y.reshape(BI, 2 * PAIR_SPAN // BI, EMB)


@functools.partial(jax.jit, static_argnames=("n_outer",))
def _tc_repack(pairs, n_outer):
    grid = n_outer // BI
    return pl.pallas_call(
        _tc_repack_block,
        grid=(grid,),
        in_specs=[pl.BlockSpec((PAIR_SPAN, 2 * EMB), lambda k: (k, 0))],
        out_specs=pl.BlockSpec((BI, 2 * PAIR_SPAN // BI, EMB),
                               lambda k: (k, 0, 0)),
        out_shape=jax.ShapeDtypeStruct((n_outer, 2 * PAIR_SPAN // BI, EMB),
                                       jnp.float32),
    )(pairs)


def kernel(atomic_numbers, embedding_table):
    total = atomic_numbers.size
    idx3 = atomic_numbers.reshape(total // (2 * PAIR_SPAN), 2,
                                  PAIR_SPAN).astype(jnp.int32)
    pidx = (idx3[:, 0, :] * VOCAB_ROWS + idx3[:, 1, :]).reshape(total // 2)
    v = embedding_table.shape[0]
    left = jnp.pad(embedding_table, ((0, 0), (0, EMB)))
    right = jnp.pad(embedding_table, ((0, 0), (EMB, 0)))
    pair_table = (left[:, None, :] + right[None, :, :]).reshape(v * v, 2 * EMB)
    pairs = _sc_embedding_gather(pair_table, pidx, total)
    return _tc_repack(pairs, atomic_numbers.shape[0])
